# baseline (speedup 1.0000x reference)
import functools

import jax
import jax.numpy as jnp
from jax import lax
from jax.experimental import pallas as pl
from jax.experimental.pallas import tpu as pltpu
from jax.experimental.pallas import tpu_sc as plsc

B = 256
C = 3
H = W = 64
HW = H * W
D = 64
V = C * 256
WROW = D // 2
NC, NS = 2, 16
NW = NC * NS
BPW = B // NW
CHUNK = 256
NCHUNK = HW // CHUNK
NPB = CHUNK // 16

_mesh = plsc.VectorSubcoreMesh(core_axis_name="c", subcore_axis_name="s")


@functools.partial(
    pl.kernel,
    mesh=_mesh,
    out_type=jax.ShapeDtypeStruct((B, D, HW), jnp.float32),
    scratch_types=[
        pltpu.VMEM((WROW * V,), jnp.int32),
        pltpu.VMEM((C, HW), jnp.int32),
        pltpu.VMEM((D, CHUNK), jnp.float32),
        pltpu.VMEM((D, CHUNK), jnp.float32),
        pltpu.SemaphoreType.DMA,
        pltpu.SemaphoreType.DMA,
    ],
    compiler_params=pltpu.CompilerParams(needs_layout_passes=False),
)
def _bow_sc(x_hbm, tw_hbm, out_hbm, table_v, x_v, o0, o1, os0, os1):
    o_b = [o0, o1]
    osem = [os0, os1]
    wid = lax.axis_index("s") * NC + lax.axis_index("c")
    pltpu.sync_copy(tw_hbm, table_v)

    himask = jnp.full((16,), -65536, jnp.int32)
    NT = BPW * NCHUNK

    def out_desc(t, j):
        b = wid * BPW + t // NCHUNK
        k = t % NCHUNK
        return pltpu.make_async_copy(
            o_b[j], out_hbm.at[b, :, pl.ds(k * CHUNK, CHUNK)], osem[j])

    def task_body(t, carry):
        k = t % NCHUNK
        for j in range(2):
            @pl.when(t % 2 == j)
            def _():
                @pl.when(k == 0)
                def _():
                    b = wid * BPW + t // NCHUNK
                    pltpu.sync_copy(x_hbm.at[b], x_v)
                @pl.when(t >= 2)
                def _():
                    out_desc(t - 2, j).wait()
                o_v = o_b[j]

                @plsc.parallel_loop(0, NPB, 1, unroll=8)
                def pb_body(pb):
                    off = k * CHUNK + pb * 16
                    i0 = x_v[0, pl.ds(off, 16)]
                    i1 = x_v[1, pl.ds(off, 16)] + 256
                    i2 = x_v[2, pl.ds(off, 16)] + 512
                    for q in range(WROW):
                        tq = table_v.at[pl.ds(q * V, V)]
                        w0 = plsc.load_gather(tq, [i0])
                        w1 = plsc.load_gather(tq, [i1])
                        w2 = plsc.load_gather(tq, [i2])
                        acc = (plsc.bitcast(w0, jnp.bfloat16)
                               + plsc.bitcast(w1, jnp.bfloat16)
                               + plsc.bitcast(w2, jnp.bfloat16))
                        accw = plsc.bitcast(acc, jnp.int32)
                        lo = plsc.bitcast(accw << 16, jnp.float32)
                        hi = plsc.bitcast(accw & himask, jnp.float32)
                        o_v[2 * q, pl.ds(pb * 16, 16)] = lo
                        o_v[2 * q + 1, pl.ds(pb * 16, 16)] = hi

                out_desc(t, j).start()
        return carry

    lax.fori_loop(0, NT, task_body, 0)
    out_desc(NT - 2, (NT - 2) % 2).wait()
    out_desc(NT - 1, (NT - 1) % 2).wait()


def kernel(x, table):
    x3 = x.reshape(B, C, HW).astype(jnp.int32)
    tb = table.astype(jnp.bfloat16).reshape(V, WROW, 2)
    tw = jax.lax.bitcast_convert_type(tb, jnp.int32)
    tw = tw.T.reshape(-1)
    out = _bow_sc(x3, tw)
    return out.reshape(B, D, H, W)


# speedup vs baseline: 1.2618x; 1.2618x over previous
import functools

import jax
import jax.numpy as jnp
from jax import lax
from jax.experimental import pallas as pl
from jax.experimental.pallas import tpu as pltpu
from jax.experimental.pallas import tpu_sc as plsc

B = 256
C = 3
H = W = 64
HW = H * W
D = 64
V = C * 256
WROW = D // 2
NC, NS = 2, 16
NW = NC * NS
BPW = B // NW
CHUNK = 512
NCHUNK = HW // CHUNK
NPB = CHUNK // 16

_mesh = plsc.VectorSubcoreMesh(core_axis_name="c", subcore_axis_name="s")


@functools.partial(
    pl.kernel,
    mesh=_mesh,
    out_type=jax.ShapeDtypeStruct((B, D, HW), jnp.float32),
    scratch_types=[
        pltpu.VMEM((WROW * V,), jnp.int32),
        pltpu.VMEM((C, HW), jnp.int32),
        pltpu.VMEM((D, CHUNK), jnp.float32),
        pltpu.VMEM((D, CHUNK), jnp.float32),
        pltpu.SemaphoreType.DMA,
        pltpu.SemaphoreType.DMA,
    ],
    compiler_params=pltpu.CompilerParams(needs_layout_passes=False),
)
def _bow_sc(x_hbm, tw_hbm, out_hbm, table_v, x_v, o0, o1, os0, os1):
    o_b = [o0, o1]
    osem = [os0, os1]
    wid = lax.axis_index("s") * NC + lax.axis_index("c")
    pltpu.sync_copy(tw_hbm, table_v)

    himask = jnp.full((16,), -65536, jnp.int32)
    NT = BPW * NCHUNK

    def out_desc(t, j):
        b = wid * BPW + t // NCHUNK
        k = t % NCHUNK
        return pltpu.make_async_copy(
            o_b[j], out_hbm.at[b, :, pl.ds(k * CHUNK, CHUNK)], osem[j])

    def task_body(t, carry):
        k = t % NCHUNK
        for j in range(2):
            @pl.when(t % 2 == j)
            def _():
                @pl.when(k == 0)
                def _():
                    b = wid * BPW + t // NCHUNK
                    pltpu.sync_copy(x_hbm.at[b], x_v)
                @pl.when(t >= 2)
                def _():
                    out_desc(t - 2, j).wait()
                o_v = o_b[j]

                @plsc.parallel_loop(0, NPB, 1, unroll=4)
                def pb_body(pb):
                    off = k * CHUNK + pb * 16
                    i0 = x_v[0, pl.ds(off, 16)]
                    i1 = x_v[1, pl.ds(off, 16)] + 256
                    i2 = x_v[2, pl.ds(off, 16)] + 512
                    for q in range(WROW):
                        tq = table_v.at[pl.ds(q * V, V)]
                        w0 = plsc.load_gather(tq, [i0])
                        w1 = plsc.load_gather(tq, [i1])
                        w2 = plsc.load_gather(tq, [i2])
                        acc = (plsc.bitcast(w0, jnp.bfloat16)
                               + plsc.bitcast(w1, jnp.bfloat16)
                               + plsc.bitcast(w2, jnp.bfloat16))
                        accw = plsc.bitcast(acc, jnp.int32)
                        lo = plsc.bitcast(accw << 16, jnp.float32)
                        hi = plsc.bitcast(accw & himask, jnp.float32)
                        o_v[2 * q, pl.ds(pb * 16, 16)] = lo
                        o_v[2 * q + 1, pl.ds(pb * 16, 16)] = hi

                out_desc(t, j).start()
        return carry

    lax.fori_loop(0, NT, task_body, 0)
    out_desc(NT - 2, (NT - 2) % 2).wait()
    out_desc(NT - 1, (NT - 1) % 2).wait()


def kernel(x, table):
    x3 = x.reshape(B, C, HW).astype(jnp.int32)
    tb = table.astype(jnp.bfloat16).reshape(V, WROW, 2)
    tw = jax.lax.bitcast_convert_type(tb, jnp.int32)
    tw = tw.T.reshape(-1)
    out = _bow_sc(x3, tw)
    return out.reshape(B, D, H, W)
